# baseline (speedup 1.0000x reference)
import functools

import jax
import jax.numpy as jnp
from jax.experimental import pallas as pl
from jax.experimental.pallas import tpu as pltpu

_B = 32
_EPS = 1e-7


def _body(idx_ref, conf_ref, shuf_ref, interp_ref, num_ref, cnt_ref, *, pt, p_total):
    i = pl.program_id(0)
    j = pl.program_id(1)

    @pl.when(jnp.logical_and(i == 0, j == 0))
    def _init():
        num_ref[...] = jnp.zeros_like(num_ref)
        cnt_ref[...] = jnp.zeros_like(cnt_ref)

    conf = conf_ref[0]
    shuf = shuf_ref[0]
    interp = interp_ref[0]

    t = conf + _EPS
    ip = interp + _EPS
    f = t * jnp.log(t / ip)
    kl_sum = jnp.sum(f, axis=1, keepdims=True)

    left = jnp.max(conf[:, 1:], axis=1, keepdims=True) > conf[:, :1]
    right = jnp.max(shuf[:, 1:], axis=1, keepdims=True) > shuf[:, :1]

    rows = jax.lax.broadcasted_iota(jnp.int32, (pt, 1), 0) + j * pt
    valid = rows < p_total
    m = jnp.logical_and(jnp.logical_and(left, jnp.logical_not(right)), valid)

    num_ref[...] += jnp.sum(jnp.where(m, kl_sum, 0.0), keepdims=True)
    cnt_ref[...] += jnp.sum(m.astype(jnp.float32), keepdims=True)


def kernel(args, lam, conf, conf_flip, loc, loc_flip, conf_shuffle,
           conf_interpolation, loc_shuffle, loc_interpolation, sup_image_index):
    B, P, C = conf.shape
    half = B // 2
    nsup = sup_image_index.shape[0]

    idx_all = jnp.concatenate(
        [sup_image_index.astype(jnp.int32),
         (sup_image_index.astype(jnp.int32) + half) % B])

    pt = 1096
    npt = pl.cdiv(P, pt)

    grid_spec = pltpu.PrefetchScalarGridSpec(
        num_scalar_prefetch=1,
        grid=(nsup, npt),
        in_specs=[
            pl.BlockSpec((1, pt, C), lambda i, j, idx: (idx[i], j, 0)),
            pl.BlockSpec((1, pt, C), lambda i, j, idx: (idx[i + nsup], j, 0)),
            pl.BlockSpec((1, pt, C), lambda i, j, idx: (idx[i], j, 0)),
        ],
        out_specs=[
            pl.BlockSpec((1, 1), lambda i, j, idx: (0, 0)),
            pl.BlockSpec((1, 1), lambda i, j, idx: (0, 0)),
        ],
    )

    num, cnt = pl.pallas_call(
        functools.partial(_body, pt=pt, p_total=P),
        grid_spec=grid_spec,
        out_shape=[
            jax.ShapeDtypeStruct((1, 1), jnp.float32),
            jax.ShapeDtypeStruct((1, 1), jnp.float32),
        ],
    )(idx_all, conf, conf_shuffle, conf_interpolation)

    count = cnt[0, 0]
    loss = jnp.where(count > 0, num[0, 0] / jnp.maximum(count, 1.0),
                     jnp.float32(0.0))
    return (jnp.zeros((1,), dtype=jnp.float32), loss)
